# baseline (speedup 1.0000x reference)
import functools

import jax
import jax.numpy as jnp
from jax import lax
from jax.experimental import pallas as pl
from jax.experimental.pallas import tpu as pltpu
from jax.experimental.pallas import tpu_sc as plsc

NC = 2
NS = 16
L = 16
NW = NC * NS

B = 16384
D = 64
BPW = B // NW
CHUNK = 128
GPC = CHUNK // L

OUT_MIN, OUT_MAX = 0.0, 5.5

_mesh = plsc.VectorSubcoreMesh(core_axis_name="c", subcore_axis_name="s",
                               num_cores=NC, num_subcores=NS)


@functools.partial(
    pl.kernel,
    out_type=jax.ShapeDtypeStruct((B,), jnp.float32),
    mesh=_mesh,
    compiler_params=pltpu.CompilerParams(
        needs_layout_passes=False, use_tc_tiling_on_sc=True),
    scratch_types=[
        pltpu.VMEM((BPW,), jnp.int32),
        pltpu.VMEM((BPW,), jnp.int32),
        pltpu.VMEM((CHUNK, D), jnp.float32),
        pltpu.VMEM((CHUNK, D), jnp.float32),
        pltpu.VMEM((CHUNK, D), jnp.float32),
        pltpu.VMEM((CHUNK, D), jnp.float32),
        pltpu.VMEM((BPW,), jnp.float32),
        pltpu.VMEM((BPW,), jnp.float32),
        pltpu.VMEM((BPW,), jnp.float32),
        pltpu.VMEM((L, L + 1), jnp.float32),
        pltpu.SemaphoreType.DMA,
        pltpu.SemaphoreType.DMA,
        pltpu.SemaphoreType.DMA,
        pltpu.SemaphoreType.DMA,
    ],
)
def _cf_kernel(users_hbm, movies_hbm, uf_hbm, ub_hbm, mf_hbm, mb_hbm,
               out_hbm, idx_u, idx_m, u_rows0, m_rows0, u_rows1, m_rows1,
               ub_v, mb_v, out_v, tmp_v, s1, s2, s3, s4):
    wid = lax.axis_index("s") * NC + lax.axis_index("c")
    base = wid * BPW

    pltpu.sync_copy(users_hbm.at[pl.ds(base, BPW)], idx_u)
    pltpu.sync_copy(movies_hbm.at[pl.ds(base, BPW)], idx_m)

    cp3 = pltpu.async_copy(ub_hbm.at[idx_u], ub_v, s3)
    cp4 = pltpu.async_copy(mb_hbm.at[idx_m], mb_v, s4)

    bufs = [(u_rows0, m_rows0), (u_rows1, m_rows1)]

    def make_issue(k, u_rows, m_rows):
        def issue(w, carry):
            vu = idx_u[pl.ds(k * CHUNK + w * L, L)]
            vm = idx_m[pl.ds(k * CHUNK + w * L, L)]
            for i in range(L):
                pltpu.async_copy(uf_hbm.at[pl.ds(vu[i], 1)],
                                 u_rows.at[pl.ds(w * L + i, 1)], s1)
                pltpu.async_copy(mf_hbm.at[pl.ds(vm[i], 1)],
                                 m_rows.at[pl.ds(w * L + i, 1)], s2)
            return carry
        return issue

    lax.fori_loop(0, GPC, make_issue(0, u_rows0, m_rows0), 0)

    for k in range(BPW // CHUNK):
        u_rows, m_rows = bufs[k % 2]
        pltpu.make_async_copy(
            uf_hbm.at[pl.ds(0, CHUNK)], u_rows, s1).wait()
        pltpu.make_async_copy(
            mf_hbm.at[pl.ds(0, CHUNK)], m_rows, s2).wait()
        if k + 1 < BPW // CHUNK:
            issue_next = make_issue(k + 1, *bufs[(k + 1) % 2])
        else:
            issue_next = None

        def group_body(g, carry):
            if issue_next is not None:
                issue_next(g, carry)
            for i in range(L):
                ri = g * L + i
                p = (u_rows[ri, pl.ds(0, L)] * m_rows[ri, pl.ds(0, L)]
                     + u_rows[ri, pl.ds(L, L)] * m_rows[ri, pl.ds(L, L)])
                q = (u_rows[ri, pl.ds(2 * L, L)] * m_rows[ri, pl.ds(2 * L, L)]
                     + u_rows[ri, pl.ds(3 * L, L)] * m_rows[ri, pl.ds(3 * L, L)])
                tmp_v[i, pl.ds(0, L)] = p + q
            lanes = lax.iota(jnp.int32, L)
            accs = [jnp.zeros((L,), jnp.float32) for _ in range(4)]
            for c in range(L):
                col = jnp.full((L,), c, jnp.int32)
                accs[c % 4] = accs[c % 4] + plsc.load_gather(tmp_v, [lanes, col])
            acc = (accs[0] + accs[1]) + (accs[2] + accs[3])
            out_v[pl.ds(k * CHUNK + g * L, L)] = acc
            return carry

        lax.fori_loop(0, GPC, group_body, 0)

    cp3.wait()
    cp4.wait()

    def final_body(g, carry):
        sl = pl.ds(g * L, L)
        acc = out_v[sl] + ub_v[sl] + mb_v[sl]
        out_v[sl] = (OUT_MAX - OUT_MIN) / (1.0 + jnp.exp(-acc)) + OUT_MIN
        return carry

    lax.fori_loop(0, BPW // L, final_body, 0)

    pltpu.sync_copy(out_v, out_hbm.at[pl.ds(base, BPW)])


def kernel(t_input, user_factors, user_bias, movie_factors, movie_bias):
    users = t_input[:, 0].astype(jnp.int32)
    movies = t_input[:, 1].astype(jnp.int32)
    n = min(user_factors.shape[0], movie_factors.shape[0])
    ufs = user_factors[:n]
    ub = user_bias[:n].reshape(-1)
    mb = movie_bias.reshape(-1)
    return _cf_kernel(users, movies, ufs, ub, movie_factors, mb)


# speedup vs baseline: 1.0130x; 1.0130x over previous
import functools

import jax
import jax.numpy as jnp
from jax import lax
from jax.experimental import pallas as pl
from jax.experimental.pallas import tpu as pltpu
from jax.experimental.pallas import tpu_sc as plsc

NC = 2
NS = 16
L = 16
NW = NC * NS

B = 16384
D = 64
BPW = B // NW
CHUNK = 128
GPC = CHUNK // L

OUT_MIN, OUT_MAX = 0.0, 5.5

_mesh = plsc.VectorSubcoreMesh(core_axis_name="c", subcore_axis_name="s",
                               num_cores=NC, num_subcores=NS)


@functools.partial(
    pl.kernel,
    out_type=jax.ShapeDtypeStruct((B,), jnp.float32),
    mesh=_mesh,
    compiler_params=pltpu.CompilerParams(
        needs_layout_passes=False, use_tc_tiling_on_sc=True),
    scratch_types=[
        pltpu.VMEM((BPW,), jnp.int32),
        pltpu.VMEM((BPW,), jnp.int32),
        pltpu.VMEM((CHUNK, D), jnp.float32),
        pltpu.VMEM((CHUNK, D), jnp.float32),
        pltpu.VMEM((CHUNK, D), jnp.float32),
        pltpu.VMEM((CHUNK, D), jnp.float32),
        pltpu.VMEM((BPW,), jnp.float32),
        pltpu.VMEM((BPW,), jnp.float32),
        pltpu.VMEM((BPW,), jnp.float32),
        pltpu.VMEM((L, L + 1), jnp.float32),
        pltpu.SemaphoreType.DMA,
        pltpu.SemaphoreType.DMA,
        pltpu.SemaphoreType.DMA,
        pltpu.SemaphoreType.DMA,
    ],
)
def _cf_kernel(users_hbm, movies_hbm, uf_hbm, ub_hbm, mf_hbm, mb_hbm,
               out_hbm, idx_u, idx_m, u_rows0, m_rows0, u_rows1, m_rows1,
               ub_v, mb_v, out_v, tmp_v, s1, s2, s3, s4):
    wid = lax.axis_index("s") * NC + lax.axis_index("c")
    base = wid * BPW

    pltpu.sync_copy(users_hbm.at[pl.ds(base, BPW)], idx_u)
    pltpu.sync_copy(movies_hbm.at[pl.ds(base, BPW)], idx_m)

    cp3 = pltpu.async_copy(ub_hbm.at[idx_u], ub_v, s3)
    cp4 = pltpu.async_copy(mb_hbm.at[idx_m], mb_v, s4)

    bufs = [(u_rows0, m_rows0), (u_rows1, m_rows1)]

    def make_issue(k, u_rows, m_rows):
        def issue(w, carry):
            vu = idx_u[pl.ds(k * CHUNK + w * L, L)]
            vm = idx_m[pl.ds(k * CHUNK + w * L, L)]
            for i in range(L):
                pltpu.async_copy(uf_hbm.at[pl.ds(vu[i], 1)],
                                 u_rows.at[pl.ds(w * L + i, 1)], s1)
                pltpu.async_copy(mf_hbm.at[pl.ds(vm[i], 1)],
                                 m_rows.at[pl.ds(w * L + i, 1)], s2)
            return carry
        return issue

    lax.fori_loop(0, GPC, make_issue(0, u_rows0, m_rows0), 0)

    for k in range(BPW // CHUNK):
        u_rows, m_rows = bufs[k % 2]
        pltpu.make_async_copy(
            uf_hbm.at[pl.ds(0, CHUNK)], u_rows, s1).wait()
        pltpu.make_async_copy(
            mf_hbm.at[pl.ds(0, CHUNK)], m_rows, s2).wait()
        if k + 1 < BPW // CHUNK:
            nu, nm = bufs[(k + 1) % 2]
            lax.fori_loop(0, GPC, make_issue(k + 1, nu, nm), 0)

        def group_body(g, carry):
            for i in range(L):
                ri = g * L + i
                p = (u_rows[ri, pl.ds(0, L)] * m_rows[ri, pl.ds(0, L)]
                     + u_rows[ri, pl.ds(L, L)] * m_rows[ri, pl.ds(L, L)])
                q = (u_rows[ri, pl.ds(2 * L, L)] * m_rows[ri, pl.ds(2 * L, L)]
                     + u_rows[ri, pl.ds(3 * L, L)] * m_rows[ri, pl.ds(3 * L, L)])
                tmp_v[i, pl.ds(0, L)] = p + q
            lanes = lax.iota(jnp.int32, L)
            accs = [jnp.zeros((L,), jnp.float32) for _ in range(4)]
            for c in range(L):
                col = jnp.full((L,), c, jnp.int32)
                accs[c % 4] = accs[c % 4] + plsc.load_gather(tmp_v, [lanes, col])
            acc = (accs[0] + accs[1]) + (accs[2] + accs[3])
            out_v[pl.ds(k * CHUNK + g * L, L)] = acc
            return carry

        lax.fori_loop(0, GPC, group_body, 0)

    cp3.wait()
    cp4.wait()

    def final_body(g, carry):
        sl = pl.ds(g * L, L)
        acc = out_v[sl] + ub_v[sl] + mb_v[sl]
        out_v[sl] = (OUT_MAX - OUT_MIN) / (1.0 + jnp.exp(-acc)) + OUT_MIN
        return carry

    lax.fori_loop(0, BPW // L, final_body, 0)

    pltpu.sync_copy(out_v, out_hbm.at[pl.ds(base, BPW)])


def kernel(t_input, user_factors, user_bias, movie_factors, movie_bias):
    users = t_input[:, 0].astype(jnp.int32)
    movies = t_input[:, 1].astype(jnp.int32)
    n = min(user_factors.shape[0], movie_factors.shape[0])
    ufs = user_factors[:n]
    ub = user_bias[:n].reshape(-1)
    mb = movie_bias.reshape(-1)
    return _cf_kernel(users, movies, ufs, ub, movie_factors, mb)
